# baseline (speedup 1.0000x reference)
import functools

import jax
import jax.numpy as jnp
from jax import lax
from jax.experimental import pallas as pl
from jax.experimental.pallas import tpu as pltpu
from jax.experimental.pallas import tpu_sc as plsc

MAX_IDX = 8191
D = 768
SCALE = float(D) ** 0.5
LANES = 16
VPR = D // LANES

NC = 2
NS = 16
NW = NC * NS

B = 4 * 8192
B_PER_W = B // NW
CHUNK = 32
N_CHUNKS = B_PER_W // CHUNK
NBUF = 4
LOOKAHEAD = 3


def _embed_body(x_hbm, table_hbm, out_hbm, idx_v,
                buf0, buf1, buf2, buf3,
                g0, g1, g2, g3, s0, s1, s2, s3):
    bufs = (buf0, buf1, buf2, buf3)
    gsems = (g0, g1, g2, g3)
    ssems = (s0, s1, s2, s3)

    wid = lax.axis_index("s") * NC + lax.axis_index("c")
    base = wid * B_PER_W

    pltpu.sync_copy(x_hbm.at[wid], idx_v)

    def clamp_body(c, carry):
        for k in range(CHUNK // LANES):
            sl = pl.ds(k * LANES, LANES)
            idx_v[c, sl] = jnp.clip(idx_v[c, sl], 0, MAX_IDX)
        return carry

    lax.fori_loop(0, N_CHUNKS, clamp_body, 0)

    def wait_dma(sem, dst_buf):
        pltpu.make_async_copy(table_hbm.at[pl.ds(0, CHUNK)], dst_buf, sem).wait()

    for j in range(LOOKAHEAD):
        pltpu.async_copy(table_hbm.at[idx_v.at[j]], bufs[j], gsems[j])

    def group_body(g, carry):
        for b in range(NBUF):
            ci = g * NBUF + b
            wait_dma(gsems[b], bufs[b])

            pltpu.async_copy(
                bufs[b], out_hbm.at[pl.ds(base + ci * CHUNK, CHUNK)], ssems[b])

            nci = ci + LOOKAHEAD
            nb = (b + LOOKAHEAD) % NBUF

            @pl.when(nci < N_CHUNKS)
            def _():
                @pl.when(nci >= NBUF)
                def _():
                    wait_dma(ssems[nb], bufs[nb])

                pltpu.async_copy(table_hbm.at[idx_v.at[nci]], bufs[nb], gsems[nb])

        return carry

    lax.fori_loop(0, N_CHUNKS // NBUF, group_body, 0)

    for b in range(NBUF):
        wait_dma(ssems[b], bufs[b])


@functools.partial(
    pl.kernel,
    out_type=jax.ShapeDtypeStruct((B, D), jnp.float32),
    mesh=plsc.VectorSubcoreMesh(core_axis_name="c", subcore_axis_name="s"),
    scratch_types=[
        pltpu.VMEM((N_CHUNKS, CHUNK), jnp.int32),
        pltpu.VMEM((CHUNK, D), jnp.float32),
        pltpu.VMEM((CHUNK, D), jnp.float32),
        pltpu.VMEM((CHUNK, D), jnp.float32),
        pltpu.VMEM((CHUNK, D), jnp.float32),
        pltpu.SemaphoreType.DMA,
        pltpu.SemaphoreType.DMA,
        pltpu.SemaphoreType.DMA,
        pltpu.SemaphoreType.DMA,
        pltpu.SemaphoreType.DMA,
        pltpu.SemaphoreType.DMA,
        pltpu.SemaphoreType.DMA,
        pltpu.SemaphoreType.DMA,
    ],
)
def _embed_kernel(x_hbm, table_hbm, out_hbm, *scratch):
    _embed_body(x_hbm, table_hbm, out_hbm, *scratch)


def _scale_body(t_ref, o_ref):
    o_ref[...] = t_ref[...] * SCALE


_scale_kernel = pl.pallas_call(
    _scale_body,
    out_shape=jax.ShapeDtypeStruct((MAX_IDX + 1, D), jnp.float32),
    grid=(8,),
    in_specs=[pl.BlockSpec(((MAX_IDX + 1) // 8, D), lambda i: (i, 0))],
    out_specs=pl.BlockSpec(((MAX_IDX + 1) // 8, D), lambda i: (i, 0)),
)


def kernel(x, table):
    scaled = _scale_kernel(table)
    out = _embed_kernel(x.reshape(NW, N_CHUNKS, CHUNK), scaled)
    return out.reshape(x.shape + (D,))


# speedup vs baseline: 1.1439x; 1.1439x over previous
import functools

import jax
import jax.numpy as jnp
from jax import lax
from jax.experimental import pallas as pl
from jax.experimental.pallas import tpu as pltpu
from jax.experimental.pallas import tpu_sc as plsc

MAX_IDX = 8191
D = 768
SCALE = float(D) ** 0.5
LANES = 16
VPR = D // LANES

NC = 2
NS = 16
NW = NC * NS

B = 4 * 8192
B_PER_W = B // NW
CHUNK = 32
N_CHUNKS = B_PER_W // CHUNK
NBUF = 4
LOOKAHEAD = 3


def _embed_body(x_hbm, table_hbm, out_hbm, idx_v,
                buf0, buf1, buf2, buf3,
                g0, g1, g2, g3, s0, s1, s2, s3):
    bufs = (buf0, buf1, buf2, buf3)
    gsems = (g0, g1, g2, g3)
    ssems = (s0, s1, s2, s3)

    wid = lax.axis_index("s") * NC + lax.axis_index("c")
    base = wid * B_PER_W

    pltpu.sync_copy(x_hbm.at[wid], idx_v)

    def clamp_body(c, carry):
        for k in range(CHUNK // LANES):
            sl = pl.ds(k * LANES, LANES)
            idx_v[c, sl] = jnp.clip(idx_v[c, sl], 0, MAX_IDX)
        return carry

    lax.fori_loop(0, N_CHUNKS, clamp_body, 0)

    def wait_dma(sem, dst_buf):
        pltpu.make_async_copy(table_hbm.at[pl.ds(0, CHUNK)], dst_buf, sem).wait()

    for j in range(LOOKAHEAD):
        pltpu.async_copy(table_hbm.at[idx_v.at[j]], bufs[j], gsems[j])

    def group_body(g, carry):
        for b in range(NBUF):
            ci = g * NBUF + b
            wait_dma(gsems[b], bufs[b])

            def row_body(r, c2):
                for j in range(VPR):
                    sl = pl.ds(j * LANES, LANES)
                    bufs[b][r, sl] = bufs[b][r, sl] * SCALE
                return c2

            lax.fori_loop(0, CHUNK, row_body, 0)

            pltpu.async_copy(
                bufs[b], out_hbm.at[pl.ds(base + ci * CHUNK, CHUNK)], ssems[b])

            nci = ci + LOOKAHEAD
            nb = (b + LOOKAHEAD) % NBUF

            @pl.when(nci < N_CHUNKS)
            def _():
                @pl.when(nci >= NBUF)
                def _():
                    wait_dma(ssems[nb], bufs[nb])

                pltpu.async_copy(table_hbm.at[idx_v.at[nci]], bufs[nb], gsems[nb])

        return carry

    lax.fori_loop(0, N_CHUNKS // NBUF, group_body, 0)

    for b in range(NBUF):
        wait_dma(ssems[b], bufs[b])


@functools.partial(
    pl.kernel,
    out_type=jax.ShapeDtypeStruct((B, D), jnp.float32),
    mesh=plsc.VectorSubcoreMesh(core_axis_name="c", subcore_axis_name="s"),
    scratch_types=[
        pltpu.VMEM((N_CHUNKS, CHUNK), jnp.int32),
        pltpu.VMEM((CHUNK, D), jnp.float32),
        pltpu.VMEM((CHUNK, D), jnp.float32),
        pltpu.VMEM((CHUNK, D), jnp.float32),
        pltpu.VMEM((CHUNK, D), jnp.float32),
        pltpu.SemaphoreType.DMA,
        pltpu.SemaphoreType.DMA,
        pltpu.SemaphoreType.DMA,
        pltpu.SemaphoreType.DMA,
        pltpu.SemaphoreType.DMA,
        pltpu.SemaphoreType.DMA,
        pltpu.SemaphoreType.DMA,
        pltpu.SemaphoreType.DMA,
    ],
)
def _embed_kernel(x_hbm, table_hbm, out_hbm, *scratch):
    _embed_body(x_hbm, table_hbm, out_hbm, *scratch)


def kernel(x, table):
    out = _embed_kernel(x.reshape(NW, N_CHUNKS, CHUNK), table)
    return out.reshape(x.shape + (D,))
